# Initial kernel scaffold; baseline (speedup 1.0000x reference)
import jax
import jax.numpy as jnp
from jax.experimental import pallas as pl


def kernel(x, edge_index, batch, W1_rel, b1, W1_root, W2_rel, b2, W2_root, W3_rel, b3, W3_root, Wl1, bl1, Wl2, bl2, Wl3, bl3):
    raise NotImplementedError("write your pallas kernel here")


# speedup vs baseline: 8.4732x; 8.4732x over previous
import functools

import jax
import jax.numpy as jnp
from jax import lax
from jax.experimental import pallas as pl
from jax.experimental.pallas import tpu as pltpu
from jax.experimental.pallas import tpu_sc as plsc

N = 10000
E = 320000
H = 128
G = 64
C = 10

NC = 2
NS = 16
NW = NC * NS
EPT = E // NW
CH = 128
NFULL = EPT // CH
REM = EPT - NFULL * CH
RPT = 624
TAIL = N - NS * RPT

ROWS = 1000
NBLK = N // ROWS


def _sc_agg_body(x_hbm, src_hbm, dst_hbm, zeros_hbm, out_hbm,
                 si0, di0, rows0, si1, di1, rows1,
                 si2, di2, rows2, acc, gsem0, gsem1, sem):
    c = lax.axis_index("c")
    s = lax.axis_index("s")
    wid = s * NC + c
    si = (si0, si1)
    di = (di0, di1)
    rows = (rows0, rows1)
    gsem = (gsem0, gsem1)

    pltpu.sync_copy(zeros_hbm.at[pl.ds(0, RPT)], acc.at[pl.ds(s * RPT, RPT)])

    @pl.when(s == NS - 1)
    def _zero_tail():
        pltpu.sync_copy(zeros_hbm.at[pl.ds(0, TAIL)],
                        acc.at[pl.ds(NS * RPT, TAIL)])

    plsc.subcore_barrier()

    base0 = wid * EPT

    for j in range(2):
        b = pl.multiple_of(base0 + j * CH, 8)
        pltpu.sync_copy(src_hbm.at[pl.ds(b, CH)], si[j])
        pltpu.sync_copy(dst_hbm.at[pl.ds(b, CH)], di[j])
        pltpu.async_copy(x_hbm.at[si[j]], rows[j], gsem[j])

    def body(i, carry):
        for j in range(2):
            k = 2 * i + j
            pltpu.make_async_copy(x_hbm.at[si[j]], rows[j], gsem[j]).wait()
            pltpu.sync_copy(rows[j], acc.at[di[j]], add=True)

            @pl.when(k + 2 < NFULL)
            def _refill():
                b = pl.multiple_of(base0 + (k + 2) * CH, 8)
                pltpu.sync_copy(src_hbm.at[pl.ds(b, CH)], si[j])
                pltpu.sync_copy(dst_hbm.at[pl.ds(b, CH)], di[j])
                pltpu.async_copy(x_hbm.at[si[j]], rows[j], gsem[j])

        return carry

    lax.fori_loop(0, NFULL // 2, body, 0)

    b = pl.multiple_of(base0 + NFULL * CH, 8)
    pltpu.sync_copy(src_hbm.at[pl.ds(b, REM)], si2)
    pltpu.sync_copy(dst_hbm.at[pl.ds(b, REM)], di2)
    pltpu.async_copy(x_hbm.at[si2], rows2, sem).wait()
    pltpu.sync_copy(rows2, acc.at[di2], add=True)

    plsc.subcore_barrier()
    pltpu.sync_copy(acc.at[pl.ds(s * RPT, RPT)],
                    out_hbm.at[c, pl.ds(s * RPT, RPT)])

    @pl.when(s == NS - 1)
    def _write_tail():
        pltpu.sync_copy(acc.at[pl.ds(NS * RPT, TAIL)],
                        out_hbm.at[c, pl.ds(NS * RPT, TAIL)])


@functools.cache
def _sc_agg_kernel():
    return functools.partial(
        pl.kernel,
        out_type=jax.ShapeDtypeStruct((NC, N, H), jnp.float32),
        mesh=plsc.VectorSubcoreMesh(
            core_axis_name="c", subcore_axis_name="s",
            num_cores=NC, num_subcores=NS),
        scratch_types=[
            pltpu.VMEM((CH,), jnp.int32),
            pltpu.VMEM((CH,), jnp.int32),
            pltpu.VMEM((CH, H), jnp.float32),
            pltpu.VMEM((CH,), jnp.int32),
            pltpu.VMEM((CH,), jnp.int32),
            pltpu.VMEM((CH, H), jnp.float32),
            pltpu.VMEM((REM,), jnp.int32),
            pltpu.VMEM((REM,), jnp.int32),
            pltpu.VMEM((REM, H), jnp.float32),
            pltpu.VMEM_SHARED((N, H), jnp.float32),
            pltpu.SemaphoreType.DMA,
            pltpu.SemaphoreType.DMA,
            pltpu.SemaphoreType.DMA,
        ],
    )(_sc_agg_body)


def _sc_agg(h_in, src, dst, zeros):
    return _sc_agg_kernel()(h_in, src, dst, zeros)


def _conv_body(a0_ref, a1_ref, x_ref, wr_ref, wo_ref, b_ref,
               brow_ref, bcol_ref, lo_ref, hi_ref,
               h_ref, gmax_ref, gsum_ref, cnt_ref):
    step = pl.program_id(0)

    @pl.when(step == 0)
    def _init():
        gmax_ref[...] = jnp.zeros_like(gmax_ref)
        gsum_ref[...] = jnp.zeros_like(gsum_ref)
        cnt_ref[...] = jnp.zeros_like(cnt_ref)

    agg = a0_ref[0] + a1_ref[0]
    h = jnp.dot(agg, wr_ref[...], preferred_element_type=jnp.float32)
    h += jnp.dot(x_ref[...], wo_ref[...], preferred_element_type=jnp.float32)
    h = jnp.maximum(h + b_ref[0:1, :], 0.0)
    h_ref[...] = h

    brow = brow_ref[0]
    giota = lax.broadcasted_iota(jnp.int32, (G, ROWS), 0)
    maskT = (giota == brow).astype(jnp.float32)
    gsum_ref[...] += jnp.dot(maskT, h, preferred_element_type=jnp.float32)
    cnt_ref[...] += jnp.dot(maskT, jnp.ones((ROWS, H), jnp.float32),
                            preferred_element_type=jnp.float32)

    bcol = bcol_ref[0]
    lo = lo_ref[step, 0]
    hi = hi_ref[step, 0]

    def gbody(g, carry):
        sel = jnp.where(bcol == g, h, 0.0)
        m = jnp.max(sel, axis=0, keepdims=True)
        cur = gmax_ref[pl.ds(g, 1), :]
        gmax_ref[pl.ds(g, 1), :] = jnp.maximum(cur, m)
        return carry

    lax.fori_loop(lo, hi + 1, gbody, 0)


def _conv_call(a2, x, w_rel, b8, w_root, brow3, bcol3, lo2, hi2):
    return pl.pallas_call(
        _conv_body,
        grid=(NBLK,),
        in_specs=[
            pl.BlockSpec((1, ROWS, H), lambda i: (0, i, 0)),
            pl.BlockSpec((1, ROWS, H), lambda i: (1, i, 0)),
            pl.BlockSpec((ROWS, H), lambda i: (i, 0)),
            pl.BlockSpec((H, H), lambda i: (0, 0)),
            pl.BlockSpec((H, H), lambda i: (0, 0)),
            pl.BlockSpec((8, H), lambda i: (0, 0)),
            pl.BlockSpec((1, 1, ROWS), lambda i: (i, 0, 0)),
            pl.BlockSpec((1, ROWS, 1), lambda i: (i, 0, 0)),
            pl.BlockSpec(memory_space=pltpu.SMEM),
            pl.BlockSpec(memory_space=pltpu.SMEM),
        ],
        out_specs=[
            pl.BlockSpec((ROWS, H), lambda i: (i, 0)),
            pl.BlockSpec((G, H), lambda i: (0, 0)),
            pl.BlockSpec((G, H), lambda i: (0, 0)),
            pl.BlockSpec((G, H), lambda i: (0, 0)),
        ],
        out_shape=[
            jax.ShapeDtypeStruct((N, H), jnp.float32),
            jax.ShapeDtypeStruct((G, H), jnp.float32),
            jax.ShapeDtypeStruct((G, H), jnp.float32),
            jax.ShapeDtypeStruct((G, H), jnp.float32),
        ],
    )(a2, a2, x, w_rel, w_root, b8, brow3, bcol3, lo2, hi2)


def _head_body(gm1, gs1, c1, gm2, gs2, c2, gm3, gs3, c3,
               a1, a2, a3, a4, a5, a6, b1_ref, w2_ref, b2_ref,
               w3_ref, b3_ref, out_ref):
    def mean(gs, cn):
        return gs[...] / jnp.maximum(cn[...], 1.0)

    f32 = jnp.float32
    t = jnp.dot(gm1[...], a1[...], preferred_element_type=f32)
    t += jnp.dot(mean(gs1, c1), a2[...], preferred_element_type=f32)
    t += jnp.dot(gm2[...], a3[...], preferred_element_type=f32)
    t += jnp.dot(mean(gs2, c2), a4[...], preferred_element_type=f32)
    t += jnp.dot(gm3[...], a5[...], preferred_element_type=f32)
    t += jnp.dot(mean(gs3, c3), a6[...], preferred_element_type=f32)
    t = jnp.maximum(t + b1_ref[0:1, :], 0.0)
    u = jnp.dot(t, w2_ref[...], preferred_element_type=f32)
    u = jnp.maximum(u + b2_ref[0:1, :], 0.0)
    v = jnp.dot(u, w3_ref[...], preferred_element_type=f32) + b3_ref[0:1, :]
    m = jnp.max(v, axis=1, keepdims=True)
    lse = jnp.log(jnp.sum(jnp.exp(v - m), axis=1, keepdims=True)) + m
    out_ref[...] = v - lse


def _head_call(args):
    return pl.pallas_call(
        _head_body,
        out_shape=jax.ShapeDtypeStruct((G, C), jnp.float32),
    )(*args)


def kernel(x, edge_index, batch, W1_rel, b1, W1_root, W2_rel, b2, W2_root,
           W3_rel, b3, W3_root, Wl1, bl1, Wl2, bl2, Wl3, bl3):
    src = edge_index[0].astype(jnp.int32)
    dst = edge_index[1].astype(jnp.int32)
    batch_i = batch.astype(jnp.int32)
    zeros = jnp.zeros((RPT, H), jnp.float32)

    brow3 = batch_i.reshape(NBLK, 1, ROWS)
    bcol3 = batch_i.reshape(NBLK, ROWS, 1)
    b2d = batch_i.reshape(NBLK, ROWS)
    lo2 = b2d[:, :1]
    hi2 = b2d[:, -1:]

    def conv(h_in, w_rel, bvec, w_root):
        acc2 = _sc_agg(h_in, src, dst, zeros)
        b8 = jnp.broadcast_to(bvec.reshape(1, H), (8, H))
        return _conv_call(acc2, h_in, w_rel, b8, w_root,
                          brow3, bcol3, lo2, hi2)

    h1, gmax1, gsum1, cnt1 = conv(x, W1_rel, b1, W1_root)
    h2, gmax2, gsum2, cnt2 = conv(h1, W2_rel, b2, W2_root)
    _, gmax3, gsum3, cnt3 = conv(h2, W3_rel, b3, W3_root)

    a_slices = [Wl1[i * H:(i + 1) * H, :] for i in range(6)]
    b1_8 = jnp.broadcast_to(bl1.reshape(1, H), (8, H))
    b2_8 = jnp.broadcast_to(bl2.reshape(1, H // 2), (8, H // 2))
    b3_8 = jnp.broadcast_to(bl3.reshape(1, C), (8, C))
    return _head_call([gmax1, gsum1, cnt1, gmax2, gsum2, cnt2,
                       gmax3, gsum3, cnt3, *a_slices,
                       b1_8, Wl2, b2_8, Wl3, b3_8])
